# Initial kernel scaffold; baseline (speedup 1.0000x reference)
import jax
import jax.numpy as jnp
from jax.experimental import pallas as pl


def kernel(x, bio_features, emb_table, W_bio, b_bio, W_head, b_head):
    raise NotImplementedError("write your pallas kernel here")


# speedup vs baseline: 17.4787x; 17.4787x over previous
import functools

import jax
import jax.numpy as jnp
from jax import lax
from jax.experimental import pallas as pl
from jax.experimental.pallas import tpu as pltpu
from jax.experimental.pallas import tpu_sc as plsc

VOCAB = 100000
D = 64
B = 4096
H = 200

NC = 2
NS = 16
NW = NC * NS
BPW = B // NW
SPLIT_A = 104
SPLIT_B = H - SPLIT_A
NBUF = 4
UNROLL = 4


def _pool_sc(x_hbm, tbl_hbm, out_hbm, idx_v, rows_v, t_v, sems):
    wid = lax.axis_index("s") * NC + lax.axis_index("c")
    base = wid * BPW

    pltpu.sync_copy(x_hbm.at[pl.ds(base * H, BPW * H)], idx_v)

    def idx_view(i, lo, n):
        return idx_v.at[pl.ds(i * H + lo, n)]

    def start(i, b):
        pltpu.async_copy(tbl_hbm.at[idx_view(i, 0, SPLIT_A)],
                         rows_v.at[b, pl.ds(0, SPLIT_A), :], sems.at[b])
        pltpu.async_copy(tbl_hbm.at[idx_view(i, SPLIT_A, SPLIT_B)],
                         rows_v.at[b, pl.ds(SPLIT_A, SPLIT_B), :], sems.at[b])

    def wait(i, b):
        pltpu.make_async_copy(tbl_hbm.at[idx_view(i, 0, SPLIT_A)],
                              rows_v.at[b, pl.ds(0, SPLIT_A), :],
                              sems.at[b]).wait()
        pltpu.make_async_copy(tbl_hbm.at[idx_view(i, SPLIT_A, SPLIT_B)],
                              rows_v.at[b, pl.ds(SPLIT_A, SPLIT_B), :],
                              sems.at[b]).wait()

    for b in range(NBUF):
        start(b, b)

    zero = jnp.zeros((16,), jnp.float32)
    scale = jnp.float32(1.0 / H)

    def outer(i0, carry):
        for b in range(NBUF):
            i = i0 * NBUF + b
            wait(i, b)

            def rbody(r, acc):
                out = []
                for c in range(D // 16):
                    a = acc[c]
                    for u in range(UNROLL):
                        a = a + rows_v[b, r * UNROLL + u, pl.ds(c * 16, 16)]
                    out.append(a)
                return tuple(out)

            acc = lax.fori_loop(0, H // UNROLL, rbody, (zero,) * (D // 16))

            @pl.when(i + NBUF < BPW)
            def _():
                start(i + NBUF, b)

            for c in range(D // 16):
                t_v[i, pl.ds(c * 16, 16)] = acc[c] * scale
        return carry

    lax.fori_loop(0, BPW // NBUF, outer, 0)

    pltpu.sync_copy(t_v, out_hbm.at[pl.ds(base, BPW), :])


@functools.partial(jax.jit, static_argnames=())
def _pool(x_flat, emb_table):
    mesh = plsc.VectorSubcoreMesh(core_axis_name="c", subcore_axis_name="s")
    f = pl.kernel(
        _pool_sc,
        mesh=mesh,
        out_type=jax.ShapeDtypeStruct((B, D), jnp.float32),
        scratch_types=[
            pltpu.VMEM((BPW * H,), jnp.int32),
            pltpu.VMEM((NBUF, H, D), jnp.float32),
            pltpu.VMEM((BPW, D), jnp.float32),
            pltpu.SemaphoreType.DMA((NBUF,)),
        ],
        compiler_params=pltpu.CompilerParams(use_tc_tiling_on_sc=False),
    )
    return f(x_flat, emb_table)


def _head_tc(t_ref, bio_ref, wb_ref, bb_ref, wh_ref, bh_ref,
             logits_ref, am_ref):
    t = t_ref[...]
    b = jnp.dot(bio_ref[...], wb_ref[...],
                preferred_element_type=jnp.float32) + bb_ref[...]
    attn = jax.nn.sigmoid(jnp.sum(t * b, axis=-1, keepdims=True))
    fused = t * attn + b * (1.0 - attn)
    logits_ref[...] = jnp.dot(fused, wh_ref[...],
                              preferred_element_type=jnp.float32) + bh_ref[...]
    am_ref[...] = jnp.mean(attn).reshape(1, 1)


def kernel(x, bio_features, emb_table, W_bio, b_bio, W_head, b_head):
    t = _pool(x.reshape(-1), emb_table)
    logits, am = pl.pallas_call(
        _head_tc,
        out_shape=(
            jax.ShapeDtypeStruct((B, 2), jnp.float32),
            jax.ShapeDtypeStruct((1, 1), jnp.float32),
        ),
    )(t, bio_features, W_bio, b_bio.reshape(1, D), W_head,
      b_head.reshape(1, 2))
    return (logits, am[0, 0])
